# baseline (speedup 1.0000x reference)
import jax
import jax.numpy as jnp
from jax.experimental import pallas as pl
from jax.experimental.pallas import tpu as pltpu

_B, _L, _K = 4, 1024, 16
_DN, _DE, _NL, _H = 10, 11, 30, 256
_NLAYERS = 4
_EPS = 1e-05
_SCALE = 30.0
_CHUNK = 128


def _mm(a, b):
    return jax.lax.dot_general(
        a, b, (((a.ndim - 1,), (0,)), ((), ())),
        precision=jax.lax.Precision.DEFAULT,
        preferred_element_type=jnp.float32)


def _ln(x, g, b):
    mu = jnp.mean(x, -1, keepdims=True)
    xc = x - mu
    var = jnp.mean(xc * xc, -1, keepdims=True)
    return xc * jax.lax.rsqrt(var + _EPS) * g + b


def _body(nsrc_ref, edges_ref, conn_ref,
          wv_ref, bv_ref,
          w1a_ref, w1c_ref, wcomb_ref, bcomb_ref,
          w2_ref, b2_ref, w3_ref, b3_ref,
          n1g_ref, n1b_ref,
          wi_ref, bi_ref, wo_ref, bo_ref,
          n2g_ref, n2b_ref,
          phiw1_ref, phib1_ref, phing_ref, phinb_ref, phiw2_ref, phib2_ref,
          rn1g_ref, rn1b_ref, rw1_ref, rb1_ref,
          rn2g_ref, rn2b_ref, rw2_ref, rb2_ref,
          out_ref, dh_ref, a_sref, c_sref, clo_sref):
    hV = _mm(nsrc_ref[0], wv_ref[...]) + bv_ref[...]

    for l in range(_NLAYERS):
        a_sref[...] = _mm(hV, w1a_ref[l])
        c = _mm(hV, w1c_ref[l])
        c_hi = c.astype(jnp.bfloat16)
        c_sref[...] = c_hi
        clo_sref[...] = (c - c_hi.astype(jnp.float32)).astype(jnp.bfloat16)
        bcomb = bcomb_ref[pl.ds(l, 1), :]
        b2 = b2_ref[pl.ds(l, 1), :]
        b3 = b3_ref[pl.ds(l, 1), :]
        w2 = w2_ref[l]
        w3 = w3_ref[l]
        wcomb = wcomb_ref[l]

        def chunk_body(ci, _):
            r0 = ci * _CHUNK
            conn_c = conn_ref[0, pl.ds(r0, _CHUNK), :]
            onehot = (conn_c[:, :, None]
                      == jax.lax.broadcasted_iota(
                          jnp.int32, (_CHUNK, _K, _L), 2)
                      ).astype(jnp.bfloat16).reshape(_CHUNK * _K, _L)
            cg = _mm(onehot, c_sref[...]) + _mm(onehot, clo_sref[...])
            e_c = jax.lax.dot_general(
                edges_ref[0, :, pl.ds(r0 * _K, _CHUNK * _K)], wcomb,
                (((0,), (0,)), ((), ())),
                precision=jax.lax.Precision.DEFAULT,
                preferred_element_type=jnp.float32)
            a_c = a_sref[pl.ds(r0, _CHUNK), :]
            pre = (cg + e_c + bcomb).reshape(_CHUNK, _K, _H) \
                + a_c[:, None, :]
            m1 = jax.nn.relu(pre).reshape(_CHUNK * _K, _H)
            m2 = jax.nn.relu(_mm(m1, w2) + b2)
            s = m2.reshape(_CHUNK, _K, _H).sum(axis=1)
            dh_ref[pl.ds(r0, _CHUNK), :] = _mm(s, w3) + _K * b3
            return 0

        jax.lax.fori_loop(0, _L // _CHUNK, chunk_body, 0)
        dh = dh_ref[...]
        hV = _ln(hV + dh / _SCALE,
                 n1g_ref[pl.ds(l, 1), :], n1b_ref[pl.ds(l, 1), :])
        ff = _mm(jax.nn.relu(_mm(hV, wi_ref[l]) + bi_ref[pl.ds(l, 1), :]),
                 wo_ref[l]) + bo_ref[pl.ds(l, 1), :]
        hV = _ln(hV + ff, n2g_ref[pl.ds(l, 1), :], n2b_ref[pl.ds(l, 1), :])

    h = jax.nn.relu(_mm(hV, phiw1_ref[...]) + phib1_ref[...])
    h = _ln(h, phing_ref[...], phinb_ref[...])
    x = jnp.sum(h * phiw2_ref[...], axis=-1) + phib2_ref[0, 0]
    x = jax.nn.relu(x)[None, :]
    x = _ln(x, rn1g_ref[...], rn1b_ref[...])
    x = jax.nn.relu(_mm(x, rw1_ref[...]) + rb1_ref[...])
    x = _ln(x, rn2g_ref[...], rn2b_ref[...])
    res = (jnp.sum(x * rw2_ref[...], axis=-1, keepdims=True)
           + rb2_ref[...] + 0.5)
    out_ref[pl.ds(pl.program_id(0), 1), :] = jnp.broadcast_to(res, (1, 128))


def kernel(nodes, edges, connections, src, node_mask, lengths, params):
    del node_mask, lengths
    p = params
    lys = p['layers']

    def stk(name):
        return jnp.stack([lp[name] for lp in lys])

    w1 = stk('W1')
    w1a = w1[:, :_H, :]
    w1b = w1[:, _H:2 * _H, :]
    w1c = w1[:, 2 * _H:, :]
    wcomb = jnp.einsum('eh,lhf->lef', p['We'], w1b)
    bcomb = jnp.einsum('h,lhf->lf', p['be'], w1b) + stk('b1')

    row = lambda v: v.reshape(1, -1)
    operands = [
        jnp.concatenate([nodes, src.reshape(_B, _L, _NL)], -1),
        edges.reshape(_B, _L * _K, _DE).transpose(0, 2, 1),
        connections.astype(jnp.int32),
        p['Wv'], row(p['bv']),
        w1a, w1c, wcomb, bcomb,
        stk('W2'), stk('b2'), stk('W3'), stk('b3'),
        stk('n1g'), stk('n1b'),
        stk('Wi'), stk('bi'), stk('Wo'), stk('bo'),
        stk('n2g'), stk('n2b'),
        p['phi_W1'], row(p['phi_b1']), row(p['phi_ng']), row(p['phi_nb']),
        p['phi_W2'].reshape(1, _H), p['phi_b2'].reshape(1, 1),
        row(p['rho_n1g']), row(p['rho_n1b']), p['rho_W1'], row(p['rho_b1']),
        row(p['rho_n2g']), row(p['rho_n2b']),
        p['rho_W2'].reshape(1, _L), p['rho_b2'].reshape(1, 1),
    ]

    specs = []
    for i, arr in enumerate(operands):
        shp = arr.shape
        if i < 3:
            specs.append(pl.BlockSpec(
                (1,) + shp[1:],
                lambda b, n=len(shp): (b,) + (0,) * (n - 1)))
        else:
            specs.append(pl.BlockSpec(
                shp, lambda b, n=len(shp): (0,) * n))

    out = pl.pallas_call(
        _body,
        grid=(_B,),
        in_specs=specs,
        out_specs=pl.BlockSpec((_B, 128), lambda b: (0, 0)),
        out_shape=jax.ShapeDtypeStruct((_B, 128), jnp.float32),
        scratch_shapes=[pltpu.VMEM((_L, _H), jnp.float32),
                        pltpu.VMEM((_L, _H), jnp.float32),
                        pltpu.VMEM((_L, _H), jnp.bfloat16),
                        pltpu.VMEM((_L, _H), jnp.bfloat16)],
        compiler_params=pltpu.CompilerParams(
            dimension_semantics=("arbitrary",)),
    )(*operands)
    return out[:, :1]


# speedup vs baseline: 1.4527x; 1.4527x over previous
import jax
import jax.numpy as jnp
from jax.experimental import pallas as pl
from jax.experimental.pallas import tpu as pltpu

_B, _L, _K = 4, 1024, 16
_DN, _DE, _NL, _H = 10, 11, 30, 256
_NLAYERS = 4
_EPS = 1e-05
_SCALE = 30.0
_CHUNK = 128


def _mm(a, b):
    return jax.lax.dot_general(
        a, b, (((a.ndim - 1,), (0,)), ((), ())),
        precision=jax.lax.Precision.DEFAULT,
        preferred_element_type=jnp.float32)


def _ln(x, g, b):
    mu = jnp.mean(x, -1, keepdims=True)
    xc = x - mu
    var = jnp.mean(xc * xc, -1, keepdims=True)
    return xc * jax.lax.rsqrt(var + _EPS) * g + b


def _body(nsrc_ref, edges_ref, conn_ref,
          wv_ref, bv_ref,
          w1a_ref, w1c_ref, wcomb_ref, bcomb_ref,
          w2_ref, b2_ref, w3_ref, b3_ref,
          n1g_ref, n1b_ref,
          wi_ref, bi_ref, wo_ref, bo_ref,
          n2g_ref, n2b_ref,
          phiw1_ref, phib1_ref, phing_ref, phinb_ref, phiw2_ref, phib2_ref,
          rn1g_ref, rn1b_ref, rw1_ref, rb1_ref,
          rn2g_ref, rn2b_ref, rw2_ref, rb2_ref,
          out_ref, dh_ref, a_sref, c_sref):
    hV = _mm(nsrc_ref[0], wv_ref[...]) + bv_ref[...]

    for l in range(_NLAYERS):
        a_sref[...] = _mm(hV, w1a_ref[l])
        c_sref[...] = _mm(hV, w1c_ref[l])
        bcomb = bcomb_ref[pl.ds(l, 1), :]
        b2 = b2_ref[pl.ds(l, 1), :]
        b3 = b3_ref[pl.ds(l, 1), :]
        w2 = w2_ref[l]
        w3 = w3_ref[l]
        wcomb = wcomb_ref[l]

        def chunk_body(ci, _):
            r0 = ci * _CHUNK
            conn_c = conn_ref[0, pl.ds(r0, _CHUNK), :]
            onehot = (conn_c[:, :, None]
                      == jax.lax.broadcasted_iota(
                          jnp.int32, (_CHUNK, _K, _L), 2)
                      ).astype(jnp.float32).reshape(_CHUNK * _K, _L)
            cg = _mm(onehot, c_sref[...])
            e_c = jax.lax.dot_general(
                edges_ref[0, :, pl.ds(r0 * _K, _CHUNK * _K)], wcomb,
                (((0,), (0,)), ((), ())),
                precision=jax.lax.Precision.DEFAULT,
                preferred_element_type=jnp.float32)
            a_c = a_sref[pl.ds(r0, _CHUNK), :]
            pre = (cg + e_c + bcomb).reshape(_CHUNK, _K, _H) \
                + a_c[:, None, :]
            m1 = jax.nn.relu(pre).reshape(_CHUNK * _K, _H)
            m2 = jax.nn.relu(_mm(m1, w2) + b2)
            s = m2.reshape(_CHUNK, _K, _H).sum(axis=1)
            dh_ref[pl.ds(r0, _CHUNK), :] = _mm(s, w3) + _K * b3
            return 0

        jax.lax.fori_loop(0, _L // _CHUNK, chunk_body, 0)
        dh = dh_ref[...]
        hV = _ln(hV + dh / _SCALE,
                 n1g_ref[pl.ds(l, 1), :], n1b_ref[pl.ds(l, 1), :])
        ff = _mm(jax.nn.relu(_mm(hV, wi_ref[l]) + bi_ref[pl.ds(l, 1), :]),
                 wo_ref[l]) + bo_ref[pl.ds(l, 1), :]
        hV = _ln(hV + ff, n2g_ref[pl.ds(l, 1), :], n2b_ref[pl.ds(l, 1), :])

    h = jax.nn.relu(_mm(hV, phiw1_ref[...]) + phib1_ref[...])
    h = _ln(h, phing_ref[...], phinb_ref[...])
    x = jnp.sum(h * phiw2_ref[...], axis=-1) + phib2_ref[0, 0]
    x = jax.nn.relu(x)[None, :]
    x = _ln(x, rn1g_ref[...], rn1b_ref[...])
    x = jax.nn.relu(_mm(x, rw1_ref[...]) + rb1_ref[...])
    x = _ln(x, rn2g_ref[...], rn2b_ref[...])
    res = (jnp.sum(x * rw2_ref[...], axis=-1, keepdims=True)
           + rb2_ref[...] + 0.5)
    out_ref[pl.ds(pl.program_id(0), 1), :] = jnp.broadcast_to(res, (1, 128))


def kernel(nodes, edges, connections, src, node_mask, lengths, params):
    del node_mask, lengths
    p = params
    lys = p['layers']

    def stk(name):
        return jnp.stack([lp[name] for lp in lys])

    w1 = stk('W1')
    w1a = w1[:, :_H, :]
    w1b = w1[:, _H:2 * _H, :]
    w1c = w1[:, 2 * _H:, :]
    wcomb = jnp.einsum('eh,lhf->lef', p['We'], w1b)
    bcomb = jnp.einsum('h,lhf->lf', p['be'], w1b) + stk('b1')

    row = lambda v: v.reshape(1, -1)
    operands = [
        jnp.concatenate([nodes, src.reshape(_B, _L, _NL)], -1),
        edges.reshape(_B, _L * _K, _DE).transpose(0, 2, 1),
        connections.astype(jnp.int32),
        p['Wv'], row(p['bv']),
        w1a, w1c, wcomb, bcomb,
        stk('W2'), stk('b2'), stk('W3'), stk('b3'),
        stk('n1g'), stk('n1b'),
        stk('Wi'), stk('bi'), stk('Wo'), stk('bo'),
        stk('n2g'), stk('n2b'),
        p['phi_W1'], row(p['phi_b1']), row(p['phi_ng']), row(p['phi_nb']),
        p['phi_W2'].reshape(1, _H), p['phi_b2'].reshape(1, 1),
        row(p['rho_n1g']), row(p['rho_n1b']), p['rho_W1'], row(p['rho_b1']),
        row(p['rho_n2g']), row(p['rho_n2b']),
        p['rho_W2'].reshape(1, _L), p['rho_b2'].reshape(1, 1),
    ]

    specs = []
    for i, arr in enumerate(operands):
        shp = arr.shape
        if i < 3:
            specs.append(pl.BlockSpec(
                (1,) + shp[1:],
                lambda b, n=len(shp): (b,) + (0,) * (n - 1)))
        else:
            specs.append(pl.BlockSpec(
                shp, lambda b, n=len(shp): (0,) * n))

    out = pl.pallas_call(
        _body,
        grid=(_B,),
        in_specs=specs,
        out_specs=pl.BlockSpec((_B, 128), lambda b: (0, 0)),
        out_shape=jax.ShapeDtypeStruct((_B, 128), jnp.float32),
        scratch_shapes=[pltpu.VMEM((_L, _H), jnp.float32),
                        pltpu.VMEM((_L, _H), jnp.float32),
                        pltpu.VMEM((_L, _H), jnp.float32)],
        compiler_params=pltpu.CompilerParams(
            dimension_semantics=("arbitrary",)),
    )(*operands)
    return out[:, :1]
